# Initial kernel scaffold; baseline (speedup 1.0000x reference)
import jax
import jax.numpy as jnp
import numpy as np
from jax.experimental import pallas as pl

E1 = 160000
T_SEQ = 8
H = 16
BLK = 2000


def _leaky(x, s=0.01):
    return jnp.where(x >= 0, x, s * x)


def _pack_gru_pair(l1, l2):
    packed = []
    for (Wi1, Wh1, bi1, bh1), (Wi2, Wh2, bi2, bh2) in zip(l1, l2):
        def cat(W1, W2):
            W1T, W2T = W1.T, W2.T
            cols = []
            for g in range(3):
                z1 = jnp.zeros_like(W2T[:, :16])
                z2 = jnp.zeros_like(W1T[:, :16])
                cols.append(jnp.concatenate([
                    jnp.concatenate([W1T[:, 16 * g:16 * g + 16], z2], axis=1),
                    jnp.concatenate([z1, W2T[:, 16 * g:16 * g + 16]], axis=1),
                ], axis=0))
            return jnp.concatenate(cols, axis=1)

        def bcat(b1, b2):
            return jnp.concatenate([b1[0:16], b2[0:16], b1[16:32], b2[16:32],
                                    b1[32:48], b2[32:48]])

        packed.append((cat(Wi1, Wi2), cat(Wh1, Wh2), bcat(bi1, bi2), bcat(bh1, bh2)))
    return packed


def _gru_edge_kernel(m1_ref, m2_ref, lc_ref, h1a_ref, h1b_ref,
                     wi0_ref, wh0_ref, bi0_ref, bh0_ref,
                     wi1_ref, wh1_ref, bi1_ref, bh1_ref,
                     p1_ref, p2_ref, b1_ref, b2_ref,
                     h1o_ref, h2o_ref, ea_ref):
    h0c = jnp.concatenate([h1a_ref[0], h1b_ref[0]], axis=1)
    h1c = jnp.concatenate([h1a_ref[1], h1b_ref[1]], axis=1)
    wi0 = wi0_ref[...]
    wh0 = wh0_ref[...]
    wi1 = wi1_ref[...]
    wh1 = wh1_ref[...]
    bi0 = bi0_ref[...]
    bh0 = bh0_ref[...]
    bi1 = bi1_ref[...]
    bh1 = bh1_ref[...]

    def gates(gi, gh, h):
        r = jax.nn.sigmoid(gi[:, 0:32] + gh[:, 0:32])
        z = jax.nn.sigmoid(gi[:, 32:64] + gh[:, 32:64])
        n = jnp.tanh(gi[:, 64:96] + r * gh[:, 64:96])
        return (1.0 - z) * n + z * h

    for t in range(T_SEQ):
        mt = jnp.concatenate([m1_ref[:, t:t + 1], m2_ref[:, t:t + 1]], axis=1)
        gi0 = jnp.dot(mt, wi0, preferred_element_type=jnp.float32) + bi0
        gh0 = jnp.dot(h0c, wh0, preferred_element_type=jnp.float32) + bh0
        h0c = gates(gi0, gh0, h0c)
        gi1 = jnp.dot(h0c, wi1, preferred_element_type=jnp.float32) + bi1
        gh1 = jnp.dot(h1c, wh1, preferred_element_type=jnp.float32) + bh1
        h1c = gates(gi1, gh1, h1c)

    h1o_ref[0] = h0c[:, 0:16]
    h1o_ref[1] = h1c[:, 0:16]
    h2o_ref[0] = h0c[:, 16:32]
    h2o_ref[1] = h1c[:, 16:32]

    f1 = jnp.concatenate([h0c[:, 0:16], h1c[:, 0:16]], axis=1)
    f2 = jnp.concatenate([h0c[:, 16:32], h1c[:, 16:32]], axis=1)
    lcw = jnp.dot(lc_ref[...], p1_ref[0:1], preferred_element_type=jnp.float32)
    ea_ref[0] = lcw + jnp.dot(f1, p1_ref[1:33], preferred_element_type=jnp.float32) + b1_ref[...]
    ea_ref[1] = jnp.dot(f2, p2_ref[...], preferred_element_type=jnp.float32) + b2_ref[...]


def _run_gru_stage(local_costs, a2s_msg, a2s_h, s2a_msg, s2a_h, params):
    m1 = a2s_msg.reshape(E1, T_SEQ)
    m2 = s2a_msg.reshape(E1, T_SEQ)
    g = _pack_gru_pair(params['gru1'], params['gru2'])
    (wi0, wh0, bi0, bh0), (wi1, wh1, bi1, bh1) = g
    p1 = params['proj1_W'].T
    p2 = params['proj2_W'].T

    grid = E1 // BLK
    full = lambda shape: pl.BlockSpec(shape, lambda i, shape=shape: tuple(0 for _ in shape))
    out_shapes = [
        jax.ShapeDtypeStruct((2, E1, H), jnp.float32),
        jax.ShapeDtypeStruct((2, E1, H), jnp.float32),
        jax.ShapeDtypeStruct((2, E1, 16), jnp.float32),
    ]
    h1, h2, ea = pl.pallas_call(
        _gru_edge_kernel,
        grid=(grid,),
        in_specs=[
            pl.BlockSpec((BLK, T_SEQ), lambda i: (i, 0)),
            pl.BlockSpec((BLK, T_SEQ), lambda i: (i, 0)),
            pl.BlockSpec((BLK, 1), lambda i: (i, 0)),
            pl.BlockSpec((2, BLK, H), lambda i: (0, i, 0)),
            pl.BlockSpec((2, BLK, H), lambda i: (0, i, 0)),
            full((2, 96)), full((32, 96)), full((96,)), full((96,)),
            full((32, 96)), full((32, 96)), full((96,)), full((96,)),
            full((33, 16)), full((32, 16)), full((16,)), full((16,)),
        ],
        out_specs=[
            pl.BlockSpec((2, BLK, H), lambda i: (0, i, 0)),
            pl.BlockSpec((2, BLK, H), lambda i: (0, i, 0)),
            pl.BlockSpec((2, BLK, 16), lambda i: (0, i, 0)),
        ],
        out_shape=out_shapes,
    )(m1, m2, local_costs, a2s_h, s2a_h,
      wi0, wh0, bi0, bh0, wi1, wh1, bi1, bh1,
      p1, p2, params['proj1_b'], params['proj2_b'])
    edge_attr = ea.reshape(2 * E1, 16)
    return h1, h2, edge_attr


def _gat_conv(x, edge_index, edge_attr, p, heads, out_ch, concat):
    N = x.shape[0]
    src = edge_index[0]
    dst = edge_index[1]
    xl = (x @ p['W'].T).reshape(N, heads, out_ch)
    el = (edge_attr @ p['W_e'].T).reshape(-1, heads, out_ch)
    a = (xl * p['att_src']).sum(-1)[src] + (xl * p['att_dst']).sum(-1)[dst] + (el * p['att_edge']).sum(-1)
    a = _leaky(a, 0.2)
    ex = jnp.exp(a)
    den = jax.ops.segment_sum(ex, dst, num_segments=N)
    out = jax.ops.segment_sum(xl[src] * ex[:, :, None], dst, num_segments=N)
    out = out / (den[:, :, None] + 1e-16)
    if concat:
        out = out.reshape(N, heads * out_ch)
    else:
        out = out.mean(axis=1)
    return out + p['bias']


def kernel(x, edge_index, local_costs, ass_to_sum_msg, ass_to_sum_hidden,
           sum_to_ass_msg, sum_to_ass_hidden, scatter_indexes,
           scatter_dom_size, neighbor_idx_info, params):
    h1, h2, edge_attr = _run_gru_stage(
        local_costs, ass_to_sum_msg, ass_to_sum_hidden,
        sum_to_ass_msg, sum_to_ass_hidden, params)

    h = _leaky(_gat_conv(x, edge_index, edge_attr, params['conv1'], 4, 8, True))
    h = _leaky(_gat_conv(h, edge_index, edge_attr, params['conv2'], 4, 8, True))
    h = _leaky(_gat_conv(h, edge_index, edge_attr, params['conv3'], 4, 8, True))
    h = _leaky(_gat_conv(h, edge_index, edge_attr, params['conv4'], 4, 32, False))

    N_DIR = 64
    pooling = jax.ops.segment_sum(h, scatter_indexes, num_segments=N_DIR + 1)[1:]
    pooling = pooling / scatter_dom_size
    alpha = pooling @ (params['attn_Wq'].T @ params['attn_Ws'][0, 0:32])
    beta = pooling @ (params['attn_Wk'].T @ params['attn_Ws'][0, 32:64])
    scores = jax.nn.sigmoid(alpha[:, None] + beta[None, :] + params['attn_bs'][0])
    idxs = neighbor_idx_info[:, 0]
    srcs = neighbor_idx_info[:, 1:]
    sel = scores.reshape(-1)[srcs + idxs[:, None] * N_DIR][:, :, None]
    w = jax.nn.softmax(sel, axis=1)
    return (w, h1, h2)


# speedup vs baseline: 15.0384x; 15.0384x over previous
import dataclasses
import functools

import jax
import jax.numpy as jnp
import numpy as np
from jax import lax
from jax.experimental import pallas as pl
from jax.experimental.pallas import tpu as pltpu
from jax.experimental.pallas import tpu_sc as plsc

E1 = 160000
EDGES = 320000
N_NODES = 10000
T_SEQ = 8
H = 16
BLK = 640
BN = 1000
NTILE = 32
CCH = 64

IL32 = np.array([(j % 4) * 8 + j // 4 for j in range(32)])
IL128 = np.array([(j % 4) * 32 + j // 4 for j in range(128)])


def _leaky(x, s=0.01):
    return jnp.where(x >= 0, x, s * x)


def _pack_gru_pair(l1, l2):
    packed = []
    for (Wi1, Wh1, bi1, bh1), (Wi2, Wh2, bi2, bh2) in zip(l1, l2):
        def cat(W1, W2):
            W1T, W2T = W1.T, W2.T
            cols = []
            for g in range(3):
                z1 = jnp.zeros_like(W2T[:, :16])
                z2 = jnp.zeros_like(W1T[:, :16])
                cols.append(jnp.concatenate([
                    jnp.concatenate([W1T[:, 16 * g:16 * g + 16], z2], axis=1),
                    jnp.concatenate([z1, W2T[:, 16 * g:16 * g + 16]], axis=1),
                ], axis=0))
            return jnp.concatenate(cols, axis=1)

        def bcat(b1, b2):
            return jnp.concatenate([b1[0:16], b2[0:16], b1[16:32], b2[16:32],
                                    b1[32:48], b2[32:48]])

        packed.append((cat(Wi1, Wi2), cat(Wh1, Wh2), bcat(bi1, bi2), bcat(bh1, bh2)))
    return packed


def _gru_edge_kernel(m1_ref, m2_ref, lc_ref, h0a_ref, h0b_ref, h1a_ref, h1b_ref,
                     wi0_ref, wh0_ref, bi0_ref, bh0_ref,
                     wi1_ref, wh1_ref, bi1_ref, bh1_ref,
                     p1_ref, p2_ref, b1_ref, b2_ref, mall_ref,
                     h1l0_ref, h1l1_ref, h2l0_ref, h2l1_ref, ae4_ref):
    h0c = jnp.concatenate([h0a_ref[...], h0b_ref[...]], axis=1)
    h1c = jnp.concatenate([h1a_ref[...], h1b_ref[...]], axis=1)
    wi0 = wi0_ref[...]
    wh0 = wh0_ref[...]
    wi1 = wi1_ref[...]
    wh1 = wh1_ref[...]
    bi0 = bi0_ref[...]
    bh0 = bh0_ref[...]
    bi1 = bi1_ref[...]
    bh1 = bh1_ref[...]

    def gates(gi, gh, h):
        r = jax.nn.sigmoid(gi[:, 0:32] + gh[:, 0:32])
        z = jax.nn.sigmoid(gi[:, 32:64] + gh[:, 32:64])
        n = jnp.tanh(gi[:, 64:96] + r * gh[:, 64:96])
        return (1.0 - z) * n + z * h

    for t in range(T_SEQ):
        gi0 = (m1_ref[:, t:t + 1] * wi0[0:1, :]
               + m2_ref[:, t:t + 1] * wi0[1:2, :]) + bi0
        gh0 = jnp.dot(h0c, wh0, preferred_element_type=jnp.float32) + bh0
        h0c = gates(gi0, gh0, h0c)
        gi1 = jnp.dot(h0c, wi1, preferred_element_type=jnp.float32) + bi1
        gh1 = jnp.dot(h1c, wh1, preferred_element_type=jnp.float32) + bh1
        h1c = gates(gi1, gh1, h1c)

    h1l0_ref[...] = h0c[:, 0:16]
    h1l1_ref[...] = h1c[:, 0:16]
    h2l0_ref[...] = h0c[:, 16:32]
    h2l1_ref[...] = h1c[:, 16:32]

    f1 = jnp.concatenate([h0c[:, 0:16], h1c[:, 0:16]], axis=1)
    f2 = jnp.concatenate([h0c[:, 16:32], h1c[:, 16:32]], axis=1)
    ea1 = (lc_ref[...] * p1_ref[0:1, :]
           + jnp.dot(f1, p1_ref[1:33], preferred_element_type=jnp.float32)
           + b1_ref[...])
    ea2 = jnp.dot(f2, p2_ref[...], preferred_element_type=jnp.float32) + b2_ref[...]
    mall = mall_ref[...]
    ae4_ref[0] = jnp.dot(ea1, mall, preferred_element_type=jnp.float32)
    ae4_ref[1] = jnp.dot(ea2, mall, preferred_element_type=jnp.float32)


def _run_gru_stage(local_costs, a2s_msg, a2s_h, s2a_msg, s2a_h, params, m_all):
    m1 = a2s_msg.reshape(E1, T_SEQ)
    m2 = s2a_msg.reshape(E1, T_SEQ)
    g = _pack_gru_pair(params['gru1'], params['gru2'])
    (wi0, wh0, bi0, bh0), (wi1, wh1, bi1, bh1) = g
    p1 = params['proj1_W'].T
    p2 = params['proj2_W'].T
    h_flat_1 = a2s_h.reshape(2 * E1, H)
    h_flat_2 = s2a_h.reshape(2 * E1, H)

    grid = E1 // BLK
    gb = E1 // BLK
    full = lambda shape: pl.BlockSpec(shape, lambda i: (0, 0))
    row2 = lambda v: v.reshape(1, -1)
    out_shapes = [jax.ShapeDtypeStruct((E1, H), jnp.float32)] * 4 + [
        jax.ShapeDtypeStruct((2, E1, 16), jnp.float32)]
    blk = lambda: pl.BlockSpec((BLK, H), lambda i: (i, 0))
    h1l0, h1l1, h2l0, h2l1, ae4 = pl.pallas_call(
        _gru_edge_kernel,
        grid=(grid,),
        in_specs=[
            pl.BlockSpec((BLK, T_SEQ), lambda i: (i, 0)),
            pl.BlockSpec((BLK, T_SEQ), lambda i: (i, 0)),
            pl.BlockSpec((BLK, 1), lambda i: (i, 0)),
            pl.BlockSpec((BLK, H), lambda i: (i, 0)),
            pl.BlockSpec((BLK, H), lambda i: (i, 0)),
            pl.BlockSpec((BLK, H), lambda i: (i + gb, 0)),
            pl.BlockSpec((BLK, H), lambda i: (i + gb, 0)),
            full((2, 96)), full((32, 96)), full((1, 96)), full((1, 96)),
            full((32, 96)), full((32, 96)), full((1, 96)), full((1, 96)),
            full((33, 16)), full((32, 16)), full((1, 16)), full((1, 16)),
            full((16, 16)),
        ],
        out_specs=[blk(), blk(), blk(), blk(),
                   pl.BlockSpec((2, BLK, 16), lambda i: (0, i, 0))],
        out_shape=out_shapes,
    )(m1, m2, local_costs, h_flat_1, h_flat_2, h_flat_1, h_flat_2,
      wi0, wh0, row2(bi0), row2(bh0), wi1, wh1, row2(bi1), row2(bh1),
      p1, p2, row2(params['proj1_b']), row2(params['proj2_b']), m_all)
    h1 = jnp.stack([h1l0, h1l1], axis=0)
    h2 = jnp.stack([h2l0, h2l1], axis=0)
    return h1, h2, ae4.reshape(EDGES, 16)


def _make_sc_conv(ch, conv_idx):
    accw = 128
    nchunks = EDGES // CCH
    kmax = (nchunks + NTILE - 1) // NTILE
    zr = 40
    nrch = N_NODES // zr
    rkmax = (nrch + 15) // 16
    mesh = plsc.VectorSubcoreMesh(core_axis_name="c", subcore_axis_name="s")
    cp = pltpu.CompilerParams()
    if "needs_layout_passes" in pltpu.CompilerParams.__dataclass_fields__:
        cp = dataclasses.replace(cp, needs_layout_passes=False)

    sw = 128

    @functools.partial(
        pl.kernel, mesh=mesh, compiler_params=cp,
        out_type=jax.ShapeDtypeStruct((2, N_NODES, accw), jnp.float32),
        scratch_types=[
            pltpu.VMEM((CCH,), jnp.int32),
            pltpu.VMEM((CCH,), jnp.int32),
            pltpu.VMEM((CCH, sw), jnp.float32),
            pltpu.VMEM((CCH, 128), jnp.float32),
            pltpu.VMEM((CCH, 16), jnp.float32),
            pltpu.VMEM((CCH, accw), jnp.float32),
            pltpu.VMEM((zr, accw), jnp.float32),
            pltpu.VMEM_SHARED((N_NODES, accw), jnp.float32),
        ],
    )
    def sc_conv(src_hbm, dst_hbm, srctab_hbm, dsttab_hbm, ae_hbm, out_hbm,
                srcv, dstv, srcbuf, dstbuf, aebuf, prodbuf, zbuf, acc):
        cid = lax.axis_index("c")
        sid = lax.axis_index("s")
        wid = sid * 2 + cid
        iota16 = lax.iota(jnp.int32, 16)
        colidx = (iota16 & 3) + (ch + 4 * conv_idx)
        zv = jnp.zeros((16,), jnp.float32)

        @pl.loop(0, zr)
        def _(r):
            for cslot in range(accw // 16):
                zbuf[r, pl.ds(cslot * 16, 16)] = zv

        @pl.loop(0, CCH)
        def _(r):
            for cslot in range(accw // 16):
                prodbuf[r, pl.ds(cslot * 16, 16)] = zv

        @pl.loop(0, rkmax)
        def _(k):
            rchunk = k * 16 + sid

            @pl.when(rchunk < nrch)
            def _():
                pltpu.sync_copy(zbuf, acc.at[pl.ds(rchunk * zr, zr)])

        plsc.subcore_barrier()

        @pl.loop(0, kmax)
        def _(k):
            chunk = k * NTILE + wid

            @pl.when(chunk < nchunks)
            def _():
                base = chunk * CCH
                pltpu.sync_copy(src_hbm.at[pl.ds(base, CCH)], srcv)
                pltpu.sync_copy(dst_hbm.at[pl.ds(base, CCH)], dstv)
                pltpu.sync_copy(srctab_hbm.at[srcv], srcbuf)
                pltpu.sync_copy(dsttab_hbm.at[dstv], dstbuf)
                pltpu.sync_copy(ae_hbm.at[pl.ds(base, CCH)], aebuf)

                @pl.loop(0, CCH)
                def _(e):
                    a = srcbuf[e, pl.ds(ch, 16)] + dstbuf[e, pl.ds(0, 16)] + aebuf[e, :]
                    ex = jnp.exp(jnp.maximum(a, 0.2 * a))
                    prodbuf[e, pl.ds(ch, 16)] = ex
                    erow = jnp.full((16,), e, jnp.int32)
                    bc = plsc.load_gather(prodbuf, [erow, colidx])
                    for half in range(ch // 16):
                        prodbuf[e, pl.ds(16 * half, 16)] = (
                            srcbuf[e, pl.ds(16 * half, 16)] * bc)

                pltpu.sync_copy(prodbuf, acc.at[dstv], add=True)

        plsc.subcore_barrier()

        @pl.loop(0, rkmax)
        def _(k):
            rchunk = k * 16 + sid

            @pl.when(rchunk < nrch)
            def _():
                r0 = rchunk * zr
                pltpu.sync_copy(acc.at[pl.ds(r0, zr)],
                                out_hbm.at[cid, pl.ds(r0, zr)])

    return sc_conv


_SC_CACHE = {}


def _sc_conv(ch, conv_idx):
    key = (ch, conv_idx)
    if key not in _SC_CACHE:
        _SC_CACHE[key] = _make_sc_conv(ch, conv_idx)
    return _SC_CACHE[key]


def _prep1_kernel(x_ref, ws_ref, wd_ref, st_o, dt_o):
    xb = x_ref[...]
    st_o[...] = jnp.dot(xb, ws_ref[...], preferred_element_type=jnp.float32)
    dt_o[...] = jnp.dot(xb, wd_ref[...], preferred_element_type=jnp.float32)


def _prep1(x, w_src, w_dst):
    grid = N_NODES // BN
    full = lambda shape: pl.BlockSpec(shape, lambda i: (0, 0))
    return pl.pallas_call(
        _prep1_kernel,
        grid=(grid,),
        in_specs=[pl.BlockSpec((BN, 128), lambda i: (i, 0)),
                  full((128, 128)), full((128, 128))],
        out_specs=[pl.BlockSpec((BN, 128), lambda i: (i, 0)),
                   pl.BlockSpec((BN, 128), lambda i: (i, 0))],
        out_shape=[jax.ShapeDtypeStruct((N_NODES, 128), jnp.float32),
                   jax.ShapeDtypeStruct((N_NODES, 128), jnp.float32)],
    )(x, w_src, w_dst)


def _make_prep_mid(conv_idx, sw_next):
    den_off = 32 + 4 * conv_idx

    def body(a0_ref, a1_ref, bias_ref, ws_ref, wd_ref, st_o, dt_o):
        num = a0_ref[:, 0:32] + a1_ref[:, 0:32]
        den = (a0_ref[:, den_off:den_off + 4] + a1_ref[:, den_off:den_off + 4])
        denb = jnp.tile(den, (1, 8))
        h = _leaky(num / (denb + 1e-16) + bias_ref[...])
        st_o[...] = jnp.dot(h, ws_ref[...], preferred_element_type=jnp.float32)
        dt_o[...] = jnp.dot(h, wd_ref[...], preferred_element_type=jnp.float32)

    def run(acc, bias_int, w_src, w_dst):
        accf = acc.reshape(2 * N_NODES, 128)
        grid = N_NODES // BN
        gb = N_NODES // BN
        full = lambda shape: pl.BlockSpec(shape, lambda i: (0, 0))
        return pl.pallas_call(
            body,
            grid=(grid,),
            in_specs=[pl.BlockSpec((BN, 128), lambda i: (i, 0)),
                      pl.BlockSpec((BN, 128), lambda i: (i + gb, 0)),
                      full((1, 32)), full((32, sw_next)), full((32, 128))],
            out_specs=[pl.BlockSpec((BN, sw_next), lambda i: (i, 0)),
                       pl.BlockSpec((BN, 128), lambda i: (i, 0))],
            out_shape=[jax.ShapeDtypeStruct((N_NODES, sw_next), jnp.float32),
                       jax.ShapeDtypeStruct((N_NODES, 128), jnp.float32)],
        )(accf, accf, bias_int, w_src, w_dst)

    return run


_prep_mid = [_make_prep_mid(0, 128), _make_prep_mid(1, 128)]


def _prep4_kernel(a0_ref, a1_ref, bias_ref, wsa_ref, wsb_ref, wd_ref,
                  sta_o, stb_o, dt_o):
    num = a0_ref[:, 0:32] + a1_ref[:, 0:32]
    den = a0_ref[:, 40:44] + a1_ref[:, 40:44]
    denb = jnp.tile(den, (1, 8))
    h = _leaky(num / (denb + 1e-16) + bias_ref[...])
    sta_o[...] = jnp.dot(h, wsa_ref[...], preferred_element_type=jnp.float32)
    stb_o[...] = jnp.dot(h, wsb_ref[...], preferred_element_type=jnp.float32)
    dt_o[...] = jnp.dot(h, wd_ref[...], preferred_element_type=jnp.float32)


def _prep4(acc, bias_int, w_src_a, w_src_b, w_dst):
    accf = acc.reshape(2 * N_NODES, 128)
    grid = N_NODES // BN
    gb = N_NODES // BN
    full = lambda shape: pl.BlockSpec(shape, lambda i: (0, 0))
    obs = lambda: pl.BlockSpec((BN, 128), lambda i: (i, 0))
    return pl.pallas_call(
        _prep4_kernel,
        grid=(grid,),
        in_specs=[pl.BlockSpec((BN, 128), lambda i: (i, 0)),
                  pl.BlockSpec((BN, 128), lambda i: (i + gb, 0)),
                  full((1, 32)), full((32, 128)), full((32, 128)),
                  full((32, 128))],
        out_specs=[obs(), obs(), obs()],
        out_shape=[jax.ShapeDtypeStruct((N_NODES, 128), jnp.float32)] * 3,
    )(accf, accf, bias_int, w_src_a, w_src_b, w_dst)


def _final_kernel(a0a_ref, a1a_ref, a0b_ref, a1b_ref, bias_ref, sa_ref, sb_ref,
                  sidx_ref, dom_ref, wq_ref, wk_ref, bs_ref, ni_ref,
                  w_ref, pool_ref):
    i = pl.program_id(0)
    numa = a0a_ref[:, 0:64] + a1a_ref[:, 0:64]
    numb = a0b_ref[:, 0:64] + a1b_ref[:, 0:64]
    den = a0a_ref[:, 76:80] + a1a_ref[:, 76:80]
    denb = jnp.tile(den, (1, 16))
    qa = numa / (denb + 1e-16)
    qb = numb / (denb + 1e-16)
    h4 = _leaky(jnp.dot(qa, sa_ref[...], preferred_element_type=jnp.float32)
                + jnp.dot(qb, sb_ref[...], preferred_element_type=jnp.float32)
                + bias_ref[...])
    sidx = sidx_ref[0]
    seg = lax.broadcasted_iota(jnp.int32, (64, BN), 0) + 1
    oh = (seg == sidx).astype(jnp.float32)
    contrib = jnp.dot(oh, h4, preferred_element_type=jnp.float32)

    @pl.when(i == 0)
    def _():
        pool_ref[...] = jnp.zeros_like(pool_ref)

    pool_ref[...] += contrib

    @pl.when(i == (N_NODES // BN) - 1)
    def _():
        poolv = pool_ref[...] / dom_ref[...]
        alpha = jnp.dot(poolv, wq_ref[...], preferred_element_type=jnp.float32)
        beta = jnp.dot(poolv, wk_ref[...], preferred_element_type=jnp.float32)
        scores = jax.nn.sigmoid(alpha + beta[:, 0][None, :] + bs_ref[0, 0])
        idxs = ni_ref[:, 0]
        cols8 = lax.broadcasted_iota(jnp.int32, (8, 64), 1)
        rsel = (idxs[:, None] == cols8).astype(jnp.float32)
        srows = jnp.dot(rsel, scores, preferred_element_type=jnp.float32)
        sels = []
        for j in range(3):
            cm = (ni_ref[:, 1 + j][:, None] == cols8).astype(jnp.float32)
            sels.append(jnp.sum(srows * cm, axis=1, keepdims=True))
        sel = jnp.concatenate(sels, axis=1)
        m = jnp.max(sel, axis=1, keepdims=True)
        e = jnp.exp(sel - m)
        w_ref[...] = e / jnp.sum(e, axis=1, keepdims=True)


def _final(acc4a, acc4b, bias4, s_a, s_b, sidx3, dom, wq, wk, bs, ni):
    accfa = acc4a.reshape(2 * N_NODES, 128)
    accfb = acc4b.reshape(2 * N_NODES, 128)
    grid = N_NODES // BN
    gb = N_NODES // BN
    full = lambda shape: pl.BlockSpec(shape, lambda i: tuple(0 for _ in shape))
    return pl.pallas_call(
        _final_kernel,
        grid=(grid,),
        in_specs=[pl.BlockSpec((BN, 128), lambda i: (i, 0)),
                  pl.BlockSpec((BN, 128), lambda i: (i + gb, 0)),
                  pl.BlockSpec((BN, 128), lambda i: (i, 0)),
                  pl.BlockSpec((BN, 128), lambda i: (i + gb, 0)),
                  full((1, 32)), full((64, 32)), full((64, 32)),
                  pl.BlockSpec((1, 1, BN), lambda i: (i, 0, 0)),
                  full((64, 1)), full((32, 1)), full((32, 1)), full((1, 1)),
                  full((8, 4))],
        out_specs=pl.BlockSpec((8, 3), lambda i: (0, 0)),
        out_shape=jax.ShapeDtypeStruct((8, 3), jnp.float32),
        scratch_shapes=[pltpu.VMEM((64, 32), jnp.float32)],
    )(accfa, accfa, accfb, accfb, bias4, s_a, s_b, sidx3, dom, wq, wk, bs, ni)


def _att_reduce(W, att, heads, out_ch):
    return jnp.einsum('hoi,ho->ih', W.reshape(heads, out_ch, W.shape[1]), att)


def _pad16(mat, off):
    z = jnp.zeros((mat.shape[0], 16), jnp.float32)
    return z.at[:, off:off + 4].set(mat)


def kernel(x, edge_index, local_costs, ass_to_sum_msg, ass_to_sum_hidden,
           sum_to_ass_msg, sum_to_ass_hidden, scatter_indexes,
           scatter_dom_size, neighbor_idx_info, params):
    il32 = jnp.asarray(IL32)
    il128 = jnp.asarray(IL128)

    convs = [params['conv1'], params['conv2'], params['conv3'], params['conv4']]
    m_cols = []
    for c, (p, out_ch) in enumerate(zip(convs, [8, 8, 8, 32])):
        m_cols.append(_att_reduce(p['W_e'], p['att_edge'], 4, out_ch))
    m_all = jnp.concatenate(m_cols, axis=1)

    w1_int = convs[0]['W'].T[:, il32]
    a1s = _pad16(_att_reduce(convs[0]['W'], convs[0]['att_src'], 4, 8), 0)
    a1d = _pad16(_att_reduce(convs[0]['W'], convs[0]['att_dst'], 4, 8), 0)
    z80 = jnp.zeros((128, 80), jnp.float32)
    z112 = jnp.zeros((128, 112), jnp.float32)
    w1_src = jnp.concatenate([w1_int, a1s, z80], axis=1)
    w1_dst = jnp.concatenate([a1d, z112], axis=1)

    mid_ws, mid_wd, mid_bias = [], [], []
    for c, (p, out_ch, ilo) in enumerate(
            [(convs[1], 8, il32), (convs[2], 8, il32), (convs[3], 32, il128)],
            start=1):
        Wp = p['W'][:, il32]
        w_int = Wp.T[:, ilo]
        a_s = _pad16(_att_reduce(Wp, p['att_src'], 4, out_ch), 4 * c)
        a_d = _pad16(_att_reduce(Wp, p['att_dst'], 4, out_ch), 4 * c)
        zd = jnp.zeros((32, 112), jnp.float32)
        mid_wd.append(jnp.concatenate([a_d, zd], axis=1))
        if c < 3:
            zs = jnp.zeros((32, 80), jnp.float32)
            mid_ws.append(jnp.concatenate([w_int, a_s, zs], axis=1))
        else:
            zs = jnp.zeros((32, 48), jnp.float32)
            mid_ws.append(jnp.concatenate([w_int[:, 0:64], a_s, zs], axis=1))
            mid_ws.append(jnp.concatenate([w_int[:, 64:128], a_s, zs], axis=1))
    for c in range(1, 3):
        mid_bias.append(convs[c]['bias'][il32].reshape(1, 32))

    h1, h2, ae4 = _run_gru_stage(
        local_costs, ass_to_sum_msg, ass_to_sum_hidden,
        sum_to_ass_msg, sum_to_ass_hidden, params, m_all)

    src = edge_index[0]
    dst = edge_index[1]

    st, dt = _prep1(x, w1_src, w1_dst)
    acc = _sc_conv(32, 0)(src, dst, st, dt, ae4)
    bias1_int = convs[0]['bias'][il32].reshape(1, 32)
    st, dt = _prep_mid[0](acc, bias1_int, mid_ws[0], mid_wd[0])
    acc = _sc_conv(32, 1)(src, dst, st, dt, ae4)
    st, dt = _prep_mid[1](acc, mid_bias[0], mid_ws[1], mid_wd[1])
    acc = _sc_conv(32, 2)(src, dst, st, dt, ae4)
    sta, stb, dt = _prep4(acc, mid_bias[1], mid_ws[2], mid_ws[3], mid_wd[2])
    acc4a = _sc_conv(64, 3)(src, dst, sta, dt, ae4)
    acc4b = _sc_conv(64, 3)(src, dst, stb, dt, ae4)

    s_np = np.zeros((128, 32), np.float32)
    for j in range(128):
        s_np[j, j // 4] = 0.25
    s_a = jnp.asarray(s_np[0:64, 0:32])
    s_b = jnp.asarray(s_np[64:128, 0:32])
    bias4 = convs[3]['bias'].reshape(1, 32)
    ws = params['attn_Ws'][0]
    wq = (params['attn_Wq'].T @ ws[0:32]).reshape(32, 1)
    wk = (params['attn_Wk'].T @ ws[32:64]).reshape(32, 1)
    bs = params['attn_bs'].reshape(1, 1)
    sidx3 = scatter_indexes.reshape(N_NODES // BN, 1, BN)

    w = _final(acc4a, acc4b, bias4, s_a, s_b, sidx3, scatter_dom_size,
               wq, wk, bs, neighbor_idx_info)
    return (w.reshape(8, 3, 1), h1, h2)
